# baseline (speedup 1.0000x reference)
import jax
import jax.numpy as jnp
from jax import lax
from jax.experimental import pallas as pl
from jax.experimental.pallas import tpu as pltpu
import jax.experimental.pallas.tpu_sc as plsc

N = 10000
E = 320000
D = 128
OUT = 128

NC = 2
NS = 16
K = 128
H = D // 2
EPT = E // NS
ETP = 20480
NCHUNK = ETP // K
NB = 4
NGRP = NCHUNK // NB
NPAD = 10240
ROWS = NPAD // NS
DEGR = NPAD // 16
TRASH = 10200


def _sc_agg_body(xT_hbm, eidx_hbm, zrow_hbm, zdeg_hbm,
                 sum_out, deg_out,
                 gidx_v, sidx_v, deg_v, acc_s, *bufs_and_sems):
    rows = list(bufs_and_sems[0:NB])
    sem_g = list(bufs_and_sems[NB:2 * NB])
    sem_s = list(bufs_and_sems[2 * NB:3 * NB])

    cid = lax.axis_index("c")
    sid = lax.axis_index("s")
    sl = pl.ds(sid * ROWS, ROWS)
    ones16 = jnp.ones((16,), jnp.float32)

    pltpu.sync_copy(eidx_hbm.at[cid, sid], gidx_v)
    pltpu.sync_copy(eidx_hbm.at[1 - cid, sid], sidx_v)

    pltpu.sync_copy(zrow_hbm, acc_s.at[sl])
    pltpu.sync_copy(zdeg_hbm, deg_v)

    def issue_gather(p, jj, b):
        pltpu.async_copy(xT_hbm.at[p].at[gidx_v.at[jj]], rows[b], sem_g[b])

    def wait_gather(b):
        pltpu.make_async_copy(xT_hbm.at[0, pl.ds(0, K)], rows[b],
                              sem_g[b]).wait()

    def issue_scatter(jj, b):
        pltpu.async_copy(rows[b], acc_s.at[sidx_v.at[jj]],
                         sem_s[b], add=True)

    def wait_scatter(b):
        pltpu.make_async_copy(rows[b], acc_s.at[pl.ds(0, K)],
                              sem_s[b]).wait()

    def count_degrees(jj):
        for u in range(K // 16):
            idx = sidx_v[jj, pl.ds(u * 16, 16)]
            plsc.addupdate_scatter(deg_v, [idx >> 4, idx & 15], ones16)

    for p in range(2):
        plsc.subcore_barrier()

        for b in range(NB):
            issue_gather(p, b, b)

        def group(g, carry):
            for b in range(NB):
                jj = g * NB + b
                bp = (b - 1) % NB
                if b == 0:
                    @pl.when(g > 0)
                    def _():
                        wait_scatter(bp)
                        issue_gather(p, jj - 1 + NB, bp)
                else:
                    wait_scatter(bp)
                    issue_gather(p, jj - 1 + NB, bp)
                wait_gather(b)
                issue_scatter(jj, b)
                if p == 0:
                    count_degrees(jj)
            return carry

        lax.fori_loop(0, NGRP - 1, group, 0)

        for b in range(NB):
            jj = (NGRP - 1) * NB + b
            bp = (b - 1) % NB
            wait_scatter(bp)
            if b == 0:
                issue_gather(p, jj - 1 + NB, bp)
            wait_gather(b)
            issue_scatter(jj, b)
            if p == 0:
                count_degrees(jj)
        wait_scatter((NCHUNK - 1) % NB)

        plsc.subcore_barrier()

        pltpu.sync_copy(acc_s.at[sl], sum_out.at[cid, p, sl])
        if p == 0:
            pltpu.sync_copy(deg_v, deg_out.at[cid, sid])
            pltpu.sync_copy(zrow_hbm, acc_s.at[sl])


def _sc_aggregate(xT, eidx4, zrow, zdeg):
    mesh = plsc.VectorSubcoreMesh(core_axis_name="c", subcore_axis_name="s")
    return pl.kernel(
        _sc_agg_body,
        out_type=(
            jax.ShapeDtypeStruct((NC, 2, NPAD, H), jnp.float32),
            jax.ShapeDtypeStruct((NC, NS, DEGR, 16), jnp.float32),
        ),
        mesh=mesh,
        compiler_params=pltpu.CompilerParams(use_tc_tiling_on_sc=False,
                                             needs_layout_passes=False),
        scratch_types=(
            [pltpu.VMEM((NCHUNK, K), jnp.int32),
             pltpu.VMEM((NCHUNK, K), jnp.int32),
             pltpu.VMEM((DEGR, 16), jnp.float32),
             pltpu.VMEM_SHARED((NPAD, H), jnp.float32)]
            + [pltpu.VMEM((K, H), jnp.float32) for _ in range(NB)]
            + [pltpu.SemaphoreType.DMA for _ in range(2 * NB)]
        ),
    )(xT, eidx4, zrow, zdeg)


def _linear_body(x_ref, fs0_ref, fs1_ref, bs0_ref, bs1_ref,
                 fd_ref, bd_ref, w_ref, b_ref, o_ref):
    x = x_ref[...]
    fd = jnp.sum(fd_ref[0], axis=0)[:, None]
    bd = jnp.sum(bd_ref[0], axis=0)[:, None]
    fr = 1.0 / jnp.maximum(fd, 1.0)
    br = 1.0 / jnp.maximum(bd, 1.0)
    h = jnp.concatenate(
        [x,
         fs0_ref[0, 0] * fr, fs1_ref[0, 0] * fr,
         bs0_ref[0, 0] * br, bs1_ref[0, 0] * br], axis=1)
    o_ref[...] = lax.dot_general(
        h, w_ref[...], (((1,), (1,)), ((), ())),
        preferred_element_type=jnp.float32) + b_ref[...]


def _linear(x, sums, degs, W, b2):
    R = 1024
    grid = (NPAD // R,)
    return pl.pallas_call(
        _linear_body,
        grid=grid,
        in_specs=[
            pl.BlockSpec((R, D), lambda i: (i, 0)),
            pl.BlockSpec((1, 1, R, H), lambda i: (0, 0, i, 0)),
            pl.BlockSpec((1, 1, R, H), lambda i: (0, 1, i, 0)),
            pl.BlockSpec((1, 1, R, H), lambda i: (1, 0, i, 0)),
            pl.BlockSpec((1, 1, R, H), lambda i: (1, 1, i, 0)),
            pl.BlockSpec((1, NS, R), lambda i: (0, 0, i)),
            pl.BlockSpec((1, NS, R), lambda i: (1, 0, i)),
            pl.BlockSpec((OUT, 3 * D), lambda i: (0, 0)),
            pl.BlockSpec((1, OUT), lambda i: (0, 0)),
        ],
        out_specs=pl.BlockSpec((R, OUT), lambda i: (i, 0)),
        out_shape=jax.ShapeDtypeStruct((N, OUT), jnp.float32),
    )(x, sums, sums, sums, sums, degs, degs, W, b2)


@jax.jit
def kernel(x, edge_index, W, b):
    xT = (jnp.pad(x, ((0, NPAD - N), (0, 0)))
          .reshape(NPAD, 2, H).transpose(1, 0, 2))
    ei3 = edge_index.reshape(2, NS, EPT)
    pad = jnp.full((2, NS, ETP - EPT), TRASH, jnp.int32)
    eidx4 = jnp.concatenate([ei3, pad], axis=2).reshape(2, NS, NCHUNK, K)
    zrow = jnp.zeros((ROWS, H), jnp.float32)
    zdeg = jnp.zeros((DEGR, 16), jnp.float32)
    sums, degs = _sc_aggregate(xT, eidx4, zrow, zdeg)
    degs3 = degs.reshape(NC, NS, NPAD)
    return _linear(x, sums, degs3, W, b.reshape(1, OUT))


# speedup vs baseline: 1.9053x; 1.9053x over previous
import jax
import jax.numpy as jnp
from jax import lax
from jax.experimental import pallas as pl
from jax.experimental.pallas import tpu as pltpu
import jax.experimental.pallas.tpu_sc as plsc

N = 10000
E = 320000
D = 128
OUT = 128

NC = 2
NS = 16
K = 80
H = D // 2
EPT = E // NS
NCHUNK = EPT // K
NB = 5
NGRP = NCHUNK // NB
NPAD = 10240
ROWS = NPAD // NS


def _sc_agg_body(xv_hbm, eidx_hbm, zrow_hbm, zdeg_hbm,
                 sum_out, deg_out,
                 gidx_v, sidx_v, deg_v, acc_s, *bufs_and_sems):
    rows = list(bufs_and_sems[0:NB])
    sem_g = list(bufs_and_sems[NB:2 * NB])
    sem_s = list(bufs_and_sems[2 * NB:3 * NB])

    cid = lax.axis_index("c")
    sid = lax.axis_index("s")
    sl = pl.ds(sid * ROWS, ROWS)
    ones16 = jnp.ones((16,), jnp.float32)

    pltpu.sync_copy(eidx_hbm.at[cid, pl.ds(sid * EPT, EPT)], gidx_v)
    pltpu.sync_copy(eidx_hbm.at[1 - cid, pl.ds(sid * EPT, EPT)], sidx_v)

    pltpu.sync_copy(zrow_hbm, acc_s.at[sl])
    pltpu.sync_copy(zdeg_hbm, deg_v)

    def dbl(k, carry):
        v = gidx_v[pl.ds(k * 16, 16)]
        gidx_v[pl.ds(k * 16, 16)] = v + v
        return carry

    lax.fori_loop(0, EPT // 16, dbl, 0)

    def wait_gather(b):
        pltpu.make_async_copy(xv_hbm.at[pl.ds(0, K)], rows[b],
                              sem_g[b]).wait()

    def issue_scatter(jj, b):
        pltpu.async_copy(rows[b], acc_s.at[sidx_v.at[pl.ds(jj * K, K)]],
                         sem_s[b], add=True)

    def wait_scatter(b):
        pltpu.make_async_copy(rows[b], acc_s.at[pl.ds(0, K)],
                              sem_s[b]).wait()

    def count_degrees(jj):
        for u in range(K // 16):
            idx = sidx_v[pl.ds(jj * K + u * 16, 16)]
            plsc.addupdate_scatter(deg_v, [idx], ones16)

    for p in range(2):
        tbl = xv_hbm if p == 0 else xv_hbm.at[pl.ds(1, 2 * N - 1)]

        def issue_gather(jj, b, tbl=tbl):
            pltpu.async_copy(tbl.at[gidx_v.at[pl.ds(jj * K, K)]],
                             rows[b], sem_g[b])

        plsc.subcore_barrier()

        for b in range(NB):
            issue_gather(b, b)

        def group(g, carry):
            for b in range(NB):
                jj = g * NB + b
                bp = (b - 1) % NB
                if b == 0:
                    @pl.when(g > 0)
                    def _():
                        wait_scatter(bp)
                        issue_gather(jj - 1 + NB, bp)
                else:
                    wait_scatter(bp)
                    issue_gather(jj - 1 + NB, bp)
                wait_gather(b)
                issue_scatter(jj, b)
                if p == 0:
                    count_degrees(jj)
            return carry

        lax.fori_loop(0, NGRP - 1, group, 0)

        for b in range(NB):
            jj = (NGRP - 1) * NB + b
            bp = (b - 1) % NB
            wait_scatter(bp)
            if b == 0:
                issue_gather(jj - 1 + NB, bp)
            wait_gather(b)
            issue_scatter(jj, b)
            if p == 0:
                count_degrees(jj)
        wait_scatter((NCHUNK - 1) % NB)

        plsc.subcore_barrier()

        pltpu.sync_copy(acc_s.at[sl], sum_out.at[cid, p, sl])
        if p == 0:
            pltpu.sync_copy(deg_v, deg_out.at[cid, sid])
            pltpu.sync_copy(zrow_hbm, acc_s.at[sl])


def _sc_aggregate(xv, eidx, zrow, zdeg):
    mesh = plsc.VectorSubcoreMesh(core_axis_name="c", subcore_axis_name="s")
    return pl.kernel(
        _sc_agg_body,
        out_type=(
            jax.ShapeDtypeStruct((NC, 2, NPAD, H), jnp.float32),
            jax.ShapeDtypeStruct((NC, NS, NPAD), jnp.float32),
        ),
        mesh=mesh,
        compiler_params=pltpu.CompilerParams(use_tc_tiling_on_sc=False,
                                             needs_layout_passes=False),
        scratch_types=(
            [pltpu.VMEM((EPT,), jnp.int32),
             pltpu.VMEM((EPT,), jnp.int32),
             pltpu.VMEM((NPAD,), jnp.float32),
             pltpu.VMEM_SHARED((NPAD, H), jnp.float32)]
            + [pltpu.VMEM((K, H), jnp.float32) for _ in range(NB)]
            + [pltpu.SemaphoreType.DMA for _ in range(2 * NB)]
        ),
    )(xv, eidx, zrow, zdeg)


def _linear_body(x_ref, fs0_ref, fs1_ref, bs0_ref, bs1_ref,
                 fd_ref, bd_ref, w_ref, b_ref, o_ref):
    x = x_ref[...]
    fd = jnp.sum(fd_ref[0], axis=0)
    bd = jnp.sum(bd_ref[0], axis=0)
    fr = 1.0 / jnp.maximum(fd, 1.0)
    br = 1.0 / jnp.maximum(bd, 1.0)
    h = jnp.concatenate(
        [x,
         fs0_ref[0, 0] * fr, fs1_ref[0, 0] * fr,
         bs0_ref[0, 0] * br, bs1_ref[0, 0] * br], axis=1)
    o_ref[...] = lax.dot_general(
        h, w_ref[...], (((1,), (1,)), ((), ())),
        preferred_element_type=jnp.float32) + b_ref[...]


def _linear(x, sums, degs, W, b2):
    R = 1024
    grid = (NPAD // R,)
    return pl.pallas_call(
        _linear_body,
        grid=grid,
        in_specs=[
            pl.BlockSpec((R, D), lambda i: (i, 0)),
            pl.BlockSpec((1, 1, R, H), lambda i: (0, 0, i, 0)),
            pl.BlockSpec((1, 1, R, H), lambda i: (0, 1, i, 0)),
            pl.BlockSpec((1, 1, R, H), lambda i: (1, 0, i, 0)),
            pl.BlockSpec((1, 1, R, H), lambda i: (1, 1, i, 0)),
            pl.BlockSpec((1, NS, R, 1), lambda i: (0, 0, i, 0)),
            pl.BlockSpec((1, NS, R, 1), lambda i: (1, 0, i, 0)),
            pl.BlockSpec((OUT, 3 * D), lambda i: (0, 0)),
            pl.BlockSpec((1, OUT), lambda i: (0, 0)),
        ],
        out_specs=pl.BlockSpec((R, OUT), lambda i: (i, 0)),
        out_shape=jax.ShapeDtypeStruct((N, OUT), jnp.float32),
    )(x, sums, sums, sums, sums, degs, degs, W, b2)


@jax.jit
def kernel(x, edge_index, W, b):
    xv = x.reshape(2 * N, H)
    zrow = jnp.zeros((ROWS, H), jnp.float32)
    zdeg = jnp.zeros((NPAD,), jnp.float32)
    sums, degs = _sc_aggregate(xv, edge_index, zrow, zdeg)
    return _linear(x, sums, degs[..., None], W, b.reshape(1, OUT))


# speedup vs baseline: 3.8119x; 2.0007x over previous
import jax
import jax.numpy as jnp
from jax import lax
from jax.experimental import pallas as pl
from jax.experimental.pallas import tpu as pltpu
import jax.experimental.pallas.tpu_sc as plsc

N = 10000
E = 320000
D = 128
OUT = 128

NC = 2
NS = 16
K = 80
H = D // 2
EPT = E // NS
NCHUNK = EPT // K
NB = 5
NGRP = NCHUNK // NB
NPAD = 10240
ROWS = NPAD // NS


def _sc_agg_body(xv_hbm, eidx_hbm, zrow_hbm, zdeg_hbm,
                 sum_out, deg_out,
                 gidx_v, sidx_v, deg_v, acc_s, *bufs_and_sems):
    rows = list(bufs_and_sems[0:NB])
    sem_g = list(bufs_and_sems[NB:2 * NB])
    sem_s = list(bufs_and_sems[2 * NB:3 * NB])

    cid = lax.axis_index("c")
    sid = lax.axis_index("s")
    sl = pl.ds(sid * ROWS, ROWS)
    ones16 = jnp.ones((16,), jnp.float32)

    pltpu.sync_copy(eidx_hbm.at[cid, pl.ds(sid * EPT, EPT)], gidx_v)
    pltpu.sync_copy(eidx_hbm.at[1 - cid, pl.ds(sid * EPT, EPT)], sidx_v)

    pltpu.sync_copy(zrow_hbm, acc_s.at[sl])
    pltpu.sync_copy(zdeg_hbm, deg_v)

    def dbl(k, carry):
        v = gidx_v[pl.ds(k * 16, 16)]
        gidx_v[pl.ds(k * 16, 16)] = v + v
        return carry

    lax.fori_loop(0, EPT // 16, dbl, 0)

    def wait_gather(b):
        pltpu.make_async_copy(xv_hbm.at[pl.ds(0, K)], rows[b],
                              sem_g[b]).wait()

    def issue_scatter(jj, b):
        pltpu.async_copy(rows[b], acc_s.at[sidx_v.at[pl.ds(jj * K, K)]],
                         sem_s[b], add=True)

    def wait_scatter(b):
        pltpu.make_async_copy(rows[b], acc_s.at[pl.ds(0, K)],
                              sem_s[b]).wait()

    def count_degrees(jj):
        for u in range(K // 16):
            idx = sidx_v[pl.ds(jj * K + u * 16, 16)]
            plsc.addupdate_scatter(deg_v, [idx], ones16)

    for p in range(2):
        tbl = xv_hbm if p == 0 else xv_hbm.at[pl.ds(1, 2 * N - 1)]

        def issue_gather(jj, b, tbl=tbl):
            pltpu.async_copy(tbl.at[gidx_v.at[pl.ds(jj * K, K)]],
                             rows[b], sem_g[b])

        plsc.subcore_barrier()

        for b in range(NB):
            issue_gather(b, b)

        def group(g, carry):
            for b in range(NB):
                jj = g * NB + b
                bp = (b - 1) % NB
                if b == 0:
                    @pl.when(g > 0)
                    def _():
                        wait_scatter(bp)
                        issue_gather(jj - 1 + NB, bp)
                else:
                    wait_scatter(bp)
                    issue_gather(jj - 1 + NB, bp)
                wait_gather(b)
                issue_scatter(jj, b)
                if p == 0:
                    count_degrees(jj)
            return carry

        lax.fori_loop(0, NGRP - 1, group, 0)

        for b in range(NB):
            jj = (NGRP - 1) * NB + b
            bp = (b - 1) % NB
            wait_scatter(bp)
            if b == 0:
                issue_gather(jj - 1 + NB, bp)
            wait_gather(b)
            issue_scatter(jj, b)
            if p == 0:
                count_degrees(jj)
        wait_scatter((NCHUNK - 1) % NB)

        plsc.subcore_barrier()

        pltpu.sync_copy(acc_s.at[sl],
                        sum_out.at[cid, sl, pl.ds(p * H, H)])
        if p == 0:
            pltpu.sync_copy(deg_v, deg_out.at[cid, sid])
            pltpu.sync_copy(zrow_hbm, acc_s.at[sl])


def _sc_aggregate(xv, eidx, zrow, zdeg):
    mesh = plsc.VectorSubcoreMesh(core_axis_name="c", subcore_axis_name="s")
    return pl.kernel(
        _sc_agg_body,
        out_type=(
            jax.ShapeDtypeStruct((NC, NPAD, D), jnp.float32),
            jax.ShapeDtypeStruct((NC, NS, NPAD), jnp.float32),
        ),
        mesh=mesh,
        compiler_params=pltpu.CompilerParams(use_tc_tiling_on_sc=False,
                                             needs_layout_passes=False),
        scratch_types=(
            [pltpu.VMEM((EPT,), jnp.int32),
             pltpu.VMEM((EPT,), jnp.int32),
             pltpu.VMEM((NPAD,), jnp.float32),
             pltpu.VMEM_SHARED((NPAD, H), jnp.float32)]
            + [pltpu.VMEM((K, H), jnp.float32) for _ in range(NB)]
            + [pltpu.SemaphoreType.DMA for _ in range(2 * NB)]
        ),
    )(xv, eidx, zrow, zdeg)


def _linear_body(x_ref, fs_ref, bs_ref,
                 fd_ref, bd_ref, w_ref, b_ref, o_ref):
    x = x_ref[...]
    fd = jnp.sum(fd_ref[0], axis=0)[:, None]
    bd = jnp.sum(bd_ref[0], axis=0)[:, None]
    fr = 1.0 / jnp.maximum(fd, 1.0)
    br = 1.0 / jnp.maximum(bd, 1.0)
    h = jnp.concatenate([x, fs_ref[0] * fr, bs_ref[0] * br], axis=1)
    o_ref[...] = lax.dot_general(
        h, w_ref[...], (((1,), (1,)), ((), ())),
        preferred_element_type=jnp.float32) + b_ref[...]


def _linear(x, sums, degs, W, b2):
    R = 1024
    grid = (NPAD // R,)
    return pl.pallas_call(
        _linear_body,
        grid=grid,
        in_specs=[
            pl.BlockSpec((R, D), lambda i: (i, 0)),
            pl.BlockSpec((1, R, D), lambda i: (0, i, 0)),
            pl.BlockSpec((1, R, D), lambda i: (1, i, 0)),
            pl.BlockSpec((1, NS, R), lambda i: (0, 0, i)),
            pl.BlockSpec((1, NS, R), lambda i: (1, 0, i)),
            pl.BlockSpec((OUT, 3 * D), lambda i: (0, 0)),
            pl.BlockSpec((1, OUT), lambda i: (0, 0)),
        ],
        out_specs=pl.BlockSpec((R, OUT), lambda i: (i, 0)),
        out_shape=jax.ShapeDtypeStruct((N, OUT), jnp.float32),
    )(x, sums, sums, degs, degs, W, b2)


@jax.jit
def kernel(x, edge_index, W, b):
    xv = x.reshape(2 * N, H)
    zrow = jnp.zeros((ROWS, H), jnp.float32)
    zdeg = jnp.zeros((NPAD,), jnp.float32)
    sums, degs = _sc_aggregate(xv, edge_index, zrow, zdeg)
    return _linear(x, sums, degs, W, b.reshape(1, OUT))
